# baseline (speedup 1.0000x reference)
import numpy as np
import jax
import jax.numpy as jnp
from jax.experimental import pallas as pl
from jax.experimental.pallas import tpu as pltpu

B = 4096
IN = 256
H = 768
D = 32
NC = 8
CPC = 128
SH = D // 2
ND = NC * D
NSH = NC * SH
BB = 512
NBLK = B // BB

_HI = jax.lax.Precision.HIGHEST


def _dot(a, b):
    return jax.lax.dot_general(a, b, (((1,), (0,)), ((), ())),
                               preferred_element_type=jnp.float32)


def _dotx(a, b):
    return jax.lax.dot_general(a, b, (((1,), (0,)), ((), ())),
                               precision=_HI, preferred_element_type=jnp.float32)


def _gelu(t):
    return t * (jax.lax.erf(t / np.sqrt(2).astype(np.float32)) + 1.0) / 2.0


def _fused_kernel(x_ref, w1_ref, b1_ref, w2_ref, b2_ref, wk_ref, bk_ref,
                  cq_ref, wv_ref, bv_ref, cb_ref, cbt_ref,
                  ws1bd_ref, bs1t_ref, ws2bd_ref, bs2t_ref,
                  kchart_ref, kcode_ref, zn_ref, ztex_ref, rw_ref, zgeo_ref,
                  vq_ref, idx_ref, znall_ref):
    x = x_ref[...]
    h1 = _gelu(_dot(x, w1_ref[...]) + b1_ref[...])
    feats = _gelu(_dot(h1, w2_ref[...]) + b2_ref[...])
    k = _dot(feats, wk_ref[...]) + bk_ref[...]
    scores = _dot(k, cq_ref[...]) / np.sqrt(float(H)).astype(np.float32)

    m = jnp.max(scores, axis=-1, keepdims=True)
    e = jnp.exp(scores - m)
    w = e / jnp.sum(e, axis=-1, keepdims=True)
    rw_ref[...] = w

    iota8 = jax.lax.broadcasted_iota(jnp.int32, (BB, NC), 1)
    wmax = jnp.max(w, axis=-1, keepdims=True)
    kchart = jnp.min(jnp.where(w == wmax, iota8, NC), axis=-1, keepdims=True)
    kchart_ref[...] = kchart

    v = _dot(feats, wv_ref[...]) + bv_ref[...]

    g = _dotx(v, cbt_ref[...])
    cn = jnp.sum(cbt_ref[...] * cbt_ref[...], axis=0)[None, :]
    t = cn - 2.0 * g
    iota128 = jax.lax.broadcasted_iota(jnp.int32, (BB, CPC), 1)

    kcode = jnp.zeros((BB, 1), dtype=jnp.int32)
    oh_parts = []
    for c in range(NC):
        tc = t[:, c * CPC:(c + 1) * CPC]
        tmin = jnp.min(tc, axis=-1, keepdims=True)
        idx_c = jnp.min(jnp.where(tc == tmin, iota128, CPC), axis=-1, keepdims=True)
        idx_ref[:, c:c + 1] = idx_c
        kcode = kcode + jnp.where(kchart == c, idx_c, 0)
        oh_parts.append(iota128 == idx_c)
    kcode_ref[...] = kcode

    onehot = jnp.concatenate(oh_parts, axis=1).astype(jnp.float32)
    zq_all = _dotx(onehot, cb_ref[...])

    rep_v = (jax.lax.broadcasted_iota(jnp.int32, (D, ND), 0)
             == jax.lax.broadcasted_iota(jnp.int32, (D, ND), 1) % D
             ).astype(jnp.float32)
    rep_w = (jax.lax.broadcasted_iota(jnp.int32, (NC, ND), 0)
             == jax.lax.broadcasted_iota(jnp.int32, (NC, ND), 1) // D
             ).astype(jnp.float32)
    tile_eye = (jax.lax.broadcasted_iota(jnp.int32, (ND, D), 0) % D
                == jax.lax.broadcasted_iota(jnp.int32, (ND, D), 1)
                ).astype(jnp.float32)
    v_tiled = _dotx(v, rep_v)
    w_rep = _dotx(w, rep_w)

    delta_all = v_tiled - zq_all
    loss = jnp.sum(delta_all * delta_all * w_rep, keepdims=True)

    hidden = _gelu(_dot(delta_all, ws1bd_ref[...]) + bs1t_ref[...])
    zn_all = _dot(hidden, ws2bd_ref[...]) + bs2t_ref[...]
    znall_ref[...] = zn_all

    zq_b = _dotx(zq_all * w_rep, tile_eye)
    zn_b = _dotx(zn_all * w_rep, tile_eye)

    zn_ref[...] = zn_b
    ztex_ref[...] = (v - zq_b) - zn_b
    zgeo_ref[...] = (v + (zq_b - v)) + zn_b

    @pl.when(pl.program_id(0) == 0)
    def _init():
        vq_ref[...] = jnp.zeros((1, 1), dtype=jnp.float32)
    vq_ref[...] += loss


def kernel(x, W1, b1, W2, b2, Wk, bk, chart_queries, Wv, bv, codebook,
           Ws1, bs1, Ws2, bs2):
    cbt = codebook.reshape(NC * CPC, D).T
    cb_bd = jnp.zeros((NC * CPC, ND), jnp.float32)
    ws1_bd = jnp.zeros((ND, NSH), jnp.float32)
    ws2_bd = jnp.zeros((NSH, ND), jnp.float32)
    for c in range(NC):
        cb_bd = cb_bd.at[c * CPC:(c + 1) * CPC, c * D:(c + 1) * D].set(codebook[c])
        ws1_bd = ws1_bd.at[c * D:(c + 1) * D, c * SH:(c + 1) * SH].set(Ws1)
        ws2_bd = ws2_bd.at[c * SH:(c + 1) * SH, c * D:(c + 1) * D].set(Ws2)
    bs1_t = jnp.tile(bs1, NC)[None, :]
    bs2_t = jnp.tile(bs2, NC)[None, :]

    full = lambda *shape: pl.BlockSpec(shape, lambda i: (0,) * len(shape))
    row = lambda *shape: pl.BlockSpec(shape, lambda i: (i,) + (0,) * (len(shape) - 1))

    out_shapes = (
        jax.ShapeDtypeStruct((B, 1), jnp.int32),
        jax.ShapeDtypeStruct((B, 1), jnp.int32),
        jax.ShapeDtypeStruct((B, D), jnp.float32),
        jax.ShapeDtypeStruct((B, D), jnp.float32),
        jax.ShapeDtypeStruct((B, NC), jnp.float32),
        jax.ShapeDtypeStruct((B, D), jnp.float32),
        jax.ShapeDtypeStruct((1, 1), jnp.float32),
        jax.ShapeDtypeStruct((B, NC), jnp.int32),
        jax.ShapeDtypeStruct((B, ND), jnp.float32),
    )
    in_specs = [
        row(BB, IN),
        full(IN, H), full(1, H), full(H, H), full(1, H), full(H, H), full(1, H),
        full(H, NC), full(H, D), full(1, D), full(NC * CPC, ND), full(D, NC * CPC),
        full(ND, NSH), full(1, NSH), full(NSH, ND), full(1, ND),
    ]
    out_specs = (
        row(BB, 1), row(BB, 1), row(BB, D), row(BB, D), row(BB, NC), row(BB, D),
        full(1, 1), row(BB, NC), row(BB, ND),
    )
    outs = pl.pallas_call(
        _fused_kernel,
        grid=(NBLK,),
        in_specs=in_specs,
        out_specs=out_specs,
        out_shape=out_shapes,
    )(x, W1, b1[None, :], W2, b2[None, :], Wk, bk[None, :],
      chart_queries.T, Wv, bv[None, :], cb_bd, cbt,
      ws1_bd, bs1_t, ws2_bd, bs2_t)

    kchart, kcode, z_n, z_tex, rw, z_geo, vq, idx, znall = outs
    vq_loss = vq[0, 0] * np.float32(1.25 / (B * D))
    return (kchart[:, 0], kcode[:, 0], z_n, z_tex, rw, z_geo, vq_loss, idx,
            znall.reshape(B, NC, D))


# speedup vs baseline: 1.3931x; 1.3931x over previous
import numpy as np
import jax
import jax.numpy as jnp
from jax.experimental import pallas as pl
from jax.experimental.pallas import tpu as pltpu

B = 4096
IN = 256
H = 768
D = 32
NC = 8
CPC = 128
SH = D // 2
ND = NC * D
NSH = NC * SH
BB = 1024
NBLK = B // BB

_VQ_SCALE = np.float32(1.25 / (B * D))


def _dot(a, b):
    return jax.lax.dot_general(a, b, (((1,), (0,)), ((), ())),
                               preferred_element_type=jnp.float32)


def _dotx(a, b):
    return jax.lax.dot_general(a, b, (((1,), (0,)), ((), ())),
                               precision=jax.lax.Precision.HIGHEST,
                               preferred_element_type=jnp.float32)


def _gelu(t):
    return t * (jax.lax.erf(t / np.sqrt(2).astype(np.float32)) + 1.0) / 2.0


def _fused_kernel(x_ref, w1_ref, b1_ref, w2_ref, b2_ref, wk_ref, bk_ref,
                  cq_ref, wv_ref, bv_ref, cb_ref, cbt_ref,
                  ws1bd_ref, bs1t_ref, ws2bd_ref, bs2t_ref,
                  kchart_ref, kcode_ref, zn_ref, ztex_ref, rw_ref, zgeo_ref,
                  vq_ref, idx_ref, znall_ref):
    x = x_ref[...]
    h1 = _gelu(_dot(x, w1_ref[...]) + b1_ref[...])
    feats = _gelu(_dot(h1, w2_ref[...]) + b2_ref[...])
    k = _dot(feats, wk_ref[...]) + bk_ref[...]
    scores = _dot(k, cq_ref[...]) / np.sqrt(float(H)).astype(np.float32)

    m = jnp.max(scores, axis=-1, keepdims=True)
    e = jnp.exp(scores - m)
    w = e / jnp.sum(e, axis=-1, keepdims=True)
    rw_ref[...] = w

    iota8 = jax.lax.broadcasted_iota(jnp.int32, (BB, NC), 1)
    wmax = jnp.max(w, axis=-1, keepdims=True)
    kchart = jnp.min(jnp.where(w == wmax, iota8, NC), axis=-1, keepdims=True)
    kchart_ref[...] = kchart

    v = _dot(feats, wv_ref[...]) + bv_ref[...]

    g = _dotx(v, cbt_ref[...])
    cbt = cbt_ref[...]
    cn = _dotx(jnp.ones((1, D), jnp.float32), cbt * cbt)
    t = cn - 2.0 * g
    iota128 = jax.lax.broadcasted_iota(jnp.int32, (BB, CPC), 1)

    kcode = jnp.zeros((BB, 1), dtype=jnp.int32)
    zq_parts = []
    for c in range(NC):
        tc = t[:, c * CPC:(c + 1) * CPC]
        tmin = jnp.min(tc, axis=-1, keepdims=True)
        idx_c = jnp.min(jnp.where(tc == tmin, iota128, CPC), axis=-1, keepdims=True)
        idx_ref[:, c:c + 1] = idx_c
        kcode = kcode + jnp.where(kchart == c, idx_c, 0)
        onehot = (iota128 == idx_c).astype(jnp.float32)
        zq_parts.append(_dotx(onehot, cb_ref[c]))
    kcode_ref[...] = kcode

    zq_all = jnp.concatenate(zq_parts, axis=1)

    rep_v = (jax.lax.broadcasted_iota(jnp.int32, (D, ND), 0)
             == jax.lax.broadcasted_iota(jnp.int32, (D, ND), 1) % D
             ).astype(jnp.float32)
    rep_w = (jax.lax.broadcasted_iota(jnp.int32, (NC, ND), 0)
             == jax.lax.broadcasted_iota(jnp.int32, (NC, ND), 1) // D
             ).astype(jnp.float32)
    tile_eye = (jax.lax.broadcasted_iota(jnp.int32, (ND, D), 0) % D
                == jax.lax.broadcasted_iota(jnp.int32, (ND, D), 1)
                ).astype(jnp.float32)
    v_tiled = _dotx(v, rep_v)
    w_rep = _dotx(w, rep_w)

    delta_all = v_tiled - zq_all
    loss = jnp.sum(delta_all * delta_all * w_rep, keepdims=True) * _VQ_SCALE

    hidden = _gelu(_dot(delta_all, ws1bd_ref[...]) + bs1t_ref[...])
    zn_all = _dot(hidden, ws2bd_ref[...]) + bs2t_ref[...]
    znall_ref[...] = zn_all

    zq_b = _dotx(zq_all * w_rep, tile_eye)
    zn_b = _dotx(zn_all * w_rep, tile_eye)

    zn_ref[...] = zn_b
    ztex_ref[...] = (v - zq_b) - zn_b
    zgeo_ref[...] = (v + (zq_b - v)) + zn_b

    @pl.when(pl.program_id(0) == 0)
    def _init():
        vq_ref[...] = jnp.zeros((1, 1), dtype=jnp.float32)
    vq_ref[...] += loss


def kernel(x, W1, b1, W2, b2, Wk, bk, chart_queries, Wv, bv, codebook,
           Ws1, bs1, Ws2, bs2):
    cbt = codebook.reshape(NC * CPC, D).T
    cnd = jnp.arange(ND)[None, :]
    rnd = jnp.arange(ND)[:, None]
    cnsh = jnp.arange(NSH)[None, :]
    ws1_bd = jnp.where(rnd // D == cnsh // SH, jnp.tile(Ws1, (NC, NC)), 0.0)
    rnsh = jnp.arange(NSH)[:, None]
    ws2_bd = jnp.where(rnsh // SH == cnd // D, jnp.tile(Ws2, (NC, NC)), 0.0)
    bs1_t = jnp.tile(bs1, NC)[None, :]
    bs2_t = jnp.tile(bs2, NC)[None, :]

    full = lambda *shape: pl.BlockSpec(shape, lambda i: (0,) * len(shape))
    row = lambda *shape: pl.BlockSpec(shape, lambda i: (i,) + (0,) * (len(shape) - 1))

    out_shapes = (
        jax.ShapeDtypeStruct((B, 1), jnp.int32),
        jax.ShapeDtypeStruct((B, 1), jnp.int32),
        jax.ShapeDtypeStruct((B, D), jnp.float32),
        jax.ShapeDtypeStruct((B, D), jnp.float32),
        jax.ShapeDtypeStruct((B, NC), jnp.float32),
        jax.ShapeDtypeStruct((B, D), jnp.float32),
        jax.ShapeDtypeStruct((1, 1), jnp.float32),
        jax.ShapeDtypeStruct((B, NC), jnp.int32),
        jax.ShapeDtypeStruct((B, ND), jnp.float32),
    )
    in_specs = [
        row(BB, IN),
        full(IN, H), full(1, H), full(H, H), full(1, H), full(H, H), full(1, H),
        full(H, NC), full(H, D), full(1, D), full(NC, CPC, D), full(D, NC * CPC),
        full(ND, NSH), full(1, NSH), full(NSH, ND), full(1, ND),
    ]
    out_specs = (
        row(BB, 1), row(BB, 1), row(BB, D), row(BB, D), row(BB, NC), row(BB, D),
        full(1, 1), row(BB, NC), row(BB, ND),
    )
    outs = pl.pallas_call(
        _fused_kernel,
        grid=(NBLK,),
        in_specs=in_specs,
        out_specs=out_specs,
        out_shape=out_shapes,
    )(x, W1, b1[None, :], W2, b2[None, :], Wk, bk[None, :],
      chart_queries.T, Wv, bv[None, :], codebook, cbt,
      ws1_bd, bs1_t, ws2_bd, bs2_t)

    kchart, kcode, z_n, z_tex, rw, z_geo, vq, idx, znall = outs
    return (kchart[:, 0], kcode[:, 0], z_n, z_tex, rw, z_geo, vq[0, 0], idx,
            znall.reshape(B, NC, D))
